# Initial kernel scaffold; baseline (speedup 1.0000x reference)
import jax
import jax.numpy as jnp
from jax.experimental import pallas as pl


def kernel(pred, target, sky_mask):
    raise NotImplementedError("write your pallas kernel here")


# speedup vs baseline: 1.2787x; 1.2787x over previous
import jax
import jax.numpy as jnp
from jax.experimental import pallas as pl
from jax.experimental.pallas import tpu as pltpu


def _mse_body(pred_ref, target_ref, mask_ref, out_ref, acc_ref):
    i = pl.program_id(0)

    @pl.when(i == 0)
    def _init():
        acc_ref[0] = 0.0
        acc_ref[1] = 0.0

    kf = 1.0 - mask_ref[0, 0].astype(jnp.float32)
    d = target_ref[0] - pred_ref[0]
    acc_ref[0] += jnp.sum(d * d * kf[None, :, :])
    acc_ref[1] += jnp.sum(kf) * 3.0

    @pl.when(i == pl.num_programs(0) - 1)
    def _fin():
        out_ref[0] = acc_ref[0] / acc_ref[1]


def kernel(pred, target, sky_mask):
    B, C, H, W = pred.shape
    out = pl.pallas_call(
        _mse_body,
        grid=(B,),
        in_specs=[
            pl.BlockSpec((1, C, H, W), lambda i: (i, 0, 0, 0)),
            pl.BlockSpec((1, C, H, W), lambda i: (i, 0, 0, 0)),
            pl.BlockSpec((1, 1, H, W), lambda i: (i, 0, 0, 0)),
        ],
        out_specs=pl.BlockSpec(memory_space=pltpu.SMEM),
        out_shape=jax.ShapeDtypeStruct((1,), jnp.float32),
        scratch_shapes=[pltpu.SMEM((2,), jnp.float32)],
    )(pred, target, sky_mask)
    return out[0]
